# Initial kernel scaffold; baseline (speedup 1.0000x reference)
import jax
import jax.numpy as jnp
from jax.experimental import pallas as pl


def kernel(x, edge_index, batch, W1, b1, W3, b3, Wg, bg):
    raise NotImplementedError("write your pallas kernel here")


# speedup vs baseline: 21.2672x; 21.2672x over previous
import functools

import jax
import jax.numpy as jnp
from jax import lax
from jax.experimental import pallas as pl
from jax.experimental.pallas import tpu as pltpu
from jax.experimental.pallas import tpu_sc as plsc

N = 10000
E = 320000
D = 128

NC = 2
NS = 16
NW = NC * NS
K = 128
N_CHUNKS = E // K
BASE_TRIPS = N_CHUNKS // NW
EXTRA = N_CHUNKS - BASE_TRIPS * NW
ROWS_MAIN = 624
ROWS_TAIL = N - ROWS_MAIN * NS
DEG_W = 16

def _worker_id():
    return lax.axis_index("s") * NC + lax.axis_index("c")


def _num_trips(w):
    return BASE_TRIPS + jnp.where(w < EXTRA, 1, 0)


@functools.cache
def _degree_kernel():
    mesh = plsc.VectorSubcoreMesh(
        core_axis_name="c", subcore_axis_name="s", num_cores=NC, num_subcores=NS)
    return functools.partial(
        pl.kernel,
        out_type=jax.ShapeDtypeStruct((NC, N, DEG_W), jnp.float32),
        mesh=mesh,
        scratch_types=[
            pltpu.VMEM_SHARED((N, DEG_W), jnp.float32),
            pltpu.VMEM((ROWS_MAIN, DEG_W), jnp.float32),
            pltpu.VMEM((K, DEG_W), jnp.float32),
            pltpu.VMEM((1, K), jnp.int32),
        ],
    )(_degree_body)


def _degree_body(dst_hbm, deg_hbm, shared_deg, zbuf, ones, idx):
    c = lax.axis_index("c")
    s = lax.axis_index("s")
    w = _worker_id()

    def fill(i, _):
        zbuf[i, :] = jnp.zeros((DEG_W,), jnp.float32)
        return 0

    lax.fori_loop(0, ROWS_MAIN, fill, 0)

    def fill1(i, _):
        ones[i, :] = jnp.full((DEG_W,), 1.0, jnp.float32)
        return 0

    lax.fori_loop(0, K, fill1, 0)

    pltpu.sync_copy(zbuf, shared_deg.at[pl.ds(s * ROWS_MAIN, ROWS_MAIN), :])

    @pl.when(s == NS - 1)
    def _():
        pltpu.sync_copy(zbuf.at[pl.ds(0, ROWS_TAIL), :],
                        shared_deg.at[pl.ds(NS * ROWS_MAIN, ROWS_TAIL), :])

    plsc.subcore_barrier()

    def body(j, _):
        base = (w + j * NW) * K
        pltpu.sync_copy(dst_hbm.at[pl.ds(base, K)], idx.at[0])
        pltpu.sync_copy(ones, shared_deg.at[idx.at[0]], add=True)
        return 0

    lax.fori_loop(0, _num_trips(w), body, 0)
    plsc.subcore_barrier()

    sl = pl.ds(s * ROWS_MAIN, ROWS_MAIN)
    pltpu.sync_copy(shared_deg.at[sl, :], deg_hbm.at[c, sl, :])

    @pl.when(s == NS - 1)
    def _():
        tl = pl.ds(NS * ROWS_MAIN, ROWS_TAIL)
        pltpu.sync_copy(shared_deg.at[tl, :], deg_hbm.at[c, tl, :])


def _linear_body(x_ref, w1_ref, w3_ref, wg_ref, b1_ref, b3_ref, deg_ref,
                 h2_ref, dis_ref, wf_ref, bf_ref):
    @pl.when(pl.program_id(0) == 0)
    def _():
        w13 = jnp.dot(w1_ref[...], w3_ref[...], preferred_element_type=jnp.float32)
        wf_ref[...] = jnp.dot(w13, wg_ref[...], preferred_element_type=jnp.float32)
        b13 = jnp.dot(b1_ref[...], w3_ref[...], preferred_element_type=jnp.float32)
        bf_ref[...] = jnp.dot(b13 + b3_ref[...], wg_ref[...],
                              preferred_element_type=jnp.float32)

    deg = deg_ref[0, :, 0:1] + deg_ref[1, :, 0:1] + 1.0
    dis = lax.rsqrt(deg)
    h = jnp.dot(x_ref[...], wf_ref[...], preferred_element_type=jnp.float32)
    h2_ref[...] = (h + bf_ref[...]) * dis
    dis_ref[...] = dis


def _linear(x, w1, w3, wg, b1, b3, deg):
    r = 2000
    g = N // r
    return pl.pallas_call(
        _linear_body,
        grid=(g,),
        in_specs=[
            pl.BlockSpec((r, D), lambda i: (i, 0)),
            pl.BlockSpec((D, D), lambda i: (0, 0)),
            pl.BlockSpec((D, D), lambda i: (0, 0)),
            pl.BlockSpec((D, D), lambda i: (0, 0)),
            pl.BlockSpec((1, D), lambda i: (0, 0)),
            pl.BlockSpec((1, D), lambda i: (0, 0)),
            pl.BlockSpec((NC, r, DEG_W), lambda i: (0, i, 0)),
        ],
        out_specs=[
            pl.BlockSpec((r, D), lambda i: (i, 0)),
            pl.BlockSpec((r, 1), lambda i: (i, 0)),
        ],
        out_shape=[
            jax.ShapeDtypeStruct((N, D), jnp.float32),
            jax.ShapeDtypeStruct((N, 1), jnp.float32),
        ],
        scratch_shapes=[
            pltpu.VMEM((D, D), jnp.float32),
            pltpu.VMEM((1, D), jnp.float32),
        ],
    )(x, w1, w3, wg, b1, b3, deg)


ZROWS = 208


@functools.cache
def _scatter_kernel():
    mesh = plsc.VectorSubcoreMesh(
        core_axis_name="c", subcore_axis_name="s", num_cores=NC, num_subcores=NS)
    return functools.partial(
        pl.kernel,
        out_type=jax.ShapeDtypeStruct((NC, N, D), jnp.float32),
        mesh=mesh,
        scratch_types=[
            pltpu.VMEM_SHARED((N, D), jnp.float32),
            pltpu.VMEM((ZROWS, D), jnp.float32),
            pltpu.VMEM((K, D), jnp.float32),
            pltpu.VMEM((1, K), jnp.int32),
            pltpu.VMEM((1, K), jnp.int32),
            pltpu.SemaphoreType.DMA,
        ],
    )(_scatter_body)


def _scatter_body(h2_hbm, src_hbm, dst_hbm, agg_hbm,
                  shared_acc, zbuf, rows, src_idx, dst_idx, sem):
    c = lax.axis_index("c")
    s = lax.axis_index("s")
    w = _worker_id()

    def fill(i, _):
        r = i // (D // 16)
        col = (i % (D // 16)) * 16
        zbuf[r, pl.ds(col, 16)] = jnp.zeros((16,), jnp.float32)
        return 0

    lax.fori_loop(0, ZROWS * (D // 16), fill, 0)

    def zero_out(i, _):
        pltpu.sync_copy(
            zbuf, shared_acc.at[pl.ds(s * ROWS_MAIN + i * ZROWS, ZROWS), :])
        return 0

    lax.fori_loop(0, ROWS_MAIN // ZROWS, zero_out, 0)

    @pl.when(s == NS - 1)
    def _():
        pltpu.sync_copy(zbuf.at[pl.ds(0, ROWS_TAIL), :],
                        shared_acc.at[pl.ds(NS * ROWS_MAIN, ROWS_TAIL), :])

    plsc.subcore_barrier()

    def body(j, _):
        base = (w + j * NW) * K
        pltpu.sync_copy(src_hbm.at[pl.ds(base, K)], src_idx.at[0])
        pltpu.sync_copy(dst_hbm.at[pl.ds(base, K)], dst_idx.at[0])
        pltpu.async_copy(h2_hbm.at[src_idx.at[0]], rows, sem).wait()
        pltpu.sync_copy(rows, shared_acc.at[dst_idx.at[0]], add=True)
        return 0

    lax.fori_loop(0, _num_trips(w), body, 0)
    plsc.subcore_barrier()

    sl = pl.ds(s * ROWS_MAIN, ROWS_MAIN)
    pltpu.sync_copy(shared_acc.at[sl, :], agg_hbm.at[c, sl, :])

    @pl.when(s == NS - 1)
    def _():
        tl = pl.ds(NS * ROWS_MAIN, ROWS_TAIL)
        pltpu.sync_copy(shared_acc.at[tl, :], agg_hbm.at[c, tl, :])


def _epilogue_body(agg_ref, h2_ref, dis_ref, bg_ref, out_ref):
    total = agg_ref[0] + agg_ref[1] + h2_ref[...]
    out_ref[...] = total * dis_ref[...] + bg_ref[...]


def _epilogue(agg, h2, dis, bg):
    r = 2000
    g = N // r
    return pl.pallas_call(
        _epilogue_body,
        grid=(g,),
        in_specs=[
            pl.BlockSpec((NC, r, D), lambda i: (0, i, 0)),
            pl.BlockSpec((r, D), lambda i: (i, 0)),
            pl.BlockSpec((r, 1), lambda i: (i, 0)),
            pl.BlockSpec((1, D), lambda i: (0, 0)),
        ],
        out_specs=pl.BlockSpec((r, D), lambda i: (i, 0)),
        out_shape=jax.ShapeDtypeStruct((N, D), jnp.float32),
    )(agg, h2, dis, bg)


def kernel(x, edge_index, batch, W1, b1, W3, b3, Wg, bg):
    del batch
    src = edge_index[0]
    dst = edge_index[1]
    deg = _degree_kernel()(dst)
    h2, dis = _linear(x, W1, W3, Wg, b1.reshape(1, D), b3.reshape(1, D), deg)
    agg = _scatter_kernel()(h2, src, dst)
    return _epilogue(agg, h2, dis, bg.reshape(1, D))
